# Initial kernel scaffold; baseline (speedup 1.0000x reference)
import jax
import jax.numpy as jnp
from jax.experimental import pallas as pl


def kernel(x1, x2, edge_index1, edge_index2, batch1, batch2, W1, b1, W2, b2, W3, b3, Watt, Wt, Wb, bt, Wfc, bfc, Wsc, bsc):
    raise NotImplementedError("write your pallas kernel here")


# speedup vs baseline: 38.2618x; 38.2618x over previous
import jax
import jax.numpy as jnp
from jax import lax
from jax.experimental import pallas as pl
from jax.experimental.pallas import tpu as pltpu
from jax.experimental.pallas import tpu_sc as plsc

NN = 10000
DD = 128
F1, F2, F3 = 64, 32, 16
KK = 16

NC, NS = 2, 16
NW = NC * NS
CHUNK = 128
NBUF = 4
GROUPS = 20
CPT = NBUF * GROUPS
EE = 320000
EPAD = NW * CPT * CHUNK
NPAD = 10240
RPT = NPAD // NS
PADROWS = 128


import functools


@functools.cache
def _mesh():
    return plsc.VectorSubcoreMesh(core_axis_name="c", subcore_axis_name="s",
                                  num_cores=NC, num_subcores=NS)


def _deg_body(dst1, dst2, zeros_h, ones_h, dp1, dp2,
              deg1, deg2, idxv, onesv, m0, m1, m2, m3):
    sems = [m0, m1, m2, m3]
    c = lax.axis_index("c")
    s = lax.axis_index("s")
    w = s * NC + c
    row0 = s * RPT
    pltpu.sync_copy(zeros_h.at[pl.ds(row0, RPT)], deg1.at[pl.ds(row0, RPT)])
    pltpu.sync_copy(zeros_h.at[pl.ds(row0, RPT)], deg2.at[pl.ds(row0, RPT)])
    pltpu.sync_copy(ones_h, onesv)
    plsc.subcore_barrier()
    for dsth, degv in ((dst1, deg1), (dst2, deg2)):
        pltpu.sync_copy(dsth.at[w], idxv)

        @pl.loop(0, GROUPS)
        def _grp(g):
            for b in range(NBUF):
                j = g * NBUF + b

                @pl.when(g > 0)
                def _drain(b=b):
                    pltpu.make_async_copy(
                        onesv, degv.at[idxv.at[0]], sems[b]).wait()

                pltpu.async_copy(onesv, degv.at[idxv.at[j]], sems[b],
                                 add=True)

        for b in range(NBUF):
            pltpu.make_async_copy(onesv, degv.at[idxv.at[0]], sems[b]).wait()
    plsc.subcore_barrier()
    pltpu.sync_copy(deg1.at[pl.ds(row0, RPT)], dp1.at[c, pl.ds(row0, RPT)])
    pltpu.sync_copy(deg2.at[pl.ds(row0, RPT)], dp2.at[c, pl.ds(row0, RPT)])


@functools.cache
def _deg_kernel():
    return pl.kernel(
        _deg_body,
        out_type=(jax.ShapeDtypeStruct((NC, NPAD), jnp.float32),) * 2,
        mesh=_mesh(),
        scratch_types=[
            pltpu.VMEM_SHARED((NPAD,), jnp.float32),
            pltpu.VMEM_SHARED((NPAD,), jnp.float32),
            pltpu.VMEM((CPT, CHUNK), jnp.int32),
            pltpu.VMEM((CHUNK,), jnp.float32),
        ] + [pltpu.SemaphoreType.DMA] * NBUF,
    )


@functools.cache
def _make_scatter(F):
    def body(hn1, src1, dst1, hn2, src2, dst2, zeros_h, out1, out2,
             acc, srcv, dstv, b0, b1, b2, b3,
             g0, g1, g2, g3, s0, s1, s2, s3):
        bufs = [b0, b1, b2, b3]
        gsems = [g0, g1, g2, g3]
        ssems = [s0, s1, s2, s3]
        c = lax.axis_index("c")
        s = lax.axis_index("s")
        w = s * NC + c
        row0 = s * RPT
        pltpu.sync_copy(zeros_h.at[pl.ds(row0, RPT)],
                        acc.at[pl.ds(row0, RPT)])
        plsc.subcore_barrier()
        for hn, srch, dsth, out in ((hn1, src1, dst1, out1),
                                    (hn2, src2, dst2, out2)):
            pltpu.sync_copy(srch.at[w], srcv)
            pltpu.sync_copy(dsth.at[w], dstv)
            for b in range(NBUF):
                pltpu.async_copy(hn.at[srcv.at[b]], bufs[b], gsems[b])

            @pl.loop(0, GROUPS)
            def _grp(g, hn=hn):
                for b in range(NBUF):
                    j = g * NBUF + b
                    pltpu.make_async_copy(
                        hn.at[srcv.at[j]], bufs[b], gsems[b]).wait()
                    pltpu.async_copy(bufs[b], acc.at[dstv.at[j]], ssems[b],
                                     add=True)

                    @pl.when(g < GROUPS - 1)
                    def _refill(b=b, j=j, hn=hn):
                        pltpu.make_async_copy(
                            bufs[b], acc.at[dstv.at[j]], ssems[b]).wait()
                        pltpu.async_copy(hn.at[srcv.at[j + NBUF]], bufs[b],
                                         gsems[b])

            for b in range(NBUF):
                jl = CPT - NBUF + b
                pltpu.make_async_copy(
                    bufs[b], acc.at[dstv.at[jl]], ssems[b]).wait()
            plsc.subcore_barrier()
            pltpu.sync_copy(acc.at[pl.ds(row0, RPT)],
                            out.at[c, pl.ds(row0, RPT)])
            pltpu.sync_copy(zeros_h.at[pl.ds(row0, RPT)],
                            acc.at[pl.ds(row0, RPT)])
            plsc.subcore_barrier()

    return pl.kernel(
        body,
        out_type=(jax.ShapeDtypeStruct((NC, NPAD, F), jnp.float32),) * 2,
        mesh=_mesh(),
        scratch_types=[
            pltpu.VMEM_SHARED((NPAD, F), jnp.float32),
            pltpu.VMEM((CPT, CHUNK), jnp.int32),
            pltpu.VMEM((CPT, CHUNK), jnp.int32),
        ] + [pltpu.VMEM((CHUNK, F), jnp.float32)] * NBUF
          + [pltpu.SemaphoreType.DMA] * (2 * NBUF),
        compiler_params=pltpu.CompilerParams(use_tc_tiling_on_sc=False),
    )


def _prep1_body(x, dt, W1_in, hn_o, dv_o):
    deg = dt[:, 0:1] + dt[:, 1:2] + 1.0
    dinv = lax.rsqrt(deg)
    h = jnp.dot(x[...], W1_in[...], preferred_element_type=jnp.float32,
                precision=lax.Precision.HIGHEST)
    hn_o[...] = h * dinv
    dv_o[...] = dinv


def _prep_mid_body(ap, hn, dv, b_in, W_in, o):
    acc = jnp.sum(ap[...], axis=0)
    z = dv[...] * (acc + hn[...]) + b_in[...]
    a = jnp.maximum(z, 0.0)
    h = jnp.dot(a, W_in[...], preferred_element_type=jnp.float32,
                precision=lax.Precision.HIGHEST)
    o[...] = h * dv[...]


def _pool_body(ap, hn, dv, mask, b3_in, Watt_in, p_o):
    z = dv[...] * (jnp.sum(ap[...], axis=0) + hn[...]) + b3_in[...]
    zm = z * mask[...]
    mean = jnp.sum(zm, axis=0, keepdims=True) * (1.0 / NN)
    tg = jnp.tanh(jnp.dot(mean, Watt_in[...],
                          preferred_element_type=jnp.float32,
                          precision=lax.Precision.HIGHEST))
    logits = lax.dot_general(z, tg, (((1,), (1,)), ((), ())),
                             preferred_element_type=jnp.float32,
                             precision=lax.Precision.HIGHEST)
    coefs = jax.nn.sigmoid(logits)
    p_o[...] = jnp.sum(coefs * zm, axis=0, keepdims=True)


def _head_body(p1_in, p2_in, Wtr_in, Wb_in, bt_in, Wfc_in, bfc_in, Wsc_in,
               bsc_in, out):
    p1 = p1_in[...]
    p2 = p2_in[...]
    Wtr = Wtr_in[...]
    scoring = jnp.zeros((1, KK), jnp.float32)
    for j in range(KK):
        wtj = Wtr[:, j * KK:(j + 1) * KK]
        scoring = scoring + jnp.dot(
            p1, wtj, preferred_element_type=jnp.float32,
            precision=lax.Precision.HIGHEST) * p2[:, j:j + 1]
    comb = jnp.concatenate([p1, p2], axis=1)
    block = lax.dot_general(comb, Wb_in[...], (((1,), (1,)), ((), ())),
                            preferred_element_type=jnp.float32,
                            precision=lax.Precision.HIGHEST)
    sc = jnp.maximum(scoring + block + bt_in[...], 0.0)
    s2 = jnp.maximum(jnp.dot(sc, Wfc_in[...],
                             preferred_element_type=jnp.float32,
                             precision=lax.Precision.HIGHEST)
                     + bfc_in[...], 0.0)
    out[...] = jax.nn.sigmoid(jnp.dot(s2, Wsc_in[...],
                                      preferred_element_type=jnp.float32,
                                      precision=lax.Precision.HIGHEST)
                              + bsc_in[...])


def _pad_rows(x):
    return jnp.concatenate(
        [x, jnp.zeros((NPAD - NN, x.shape[1]), x.dtype)], axis=0)


def _prep_edges(ei):
    pad = EPAD - EE
    pad_idx = NN + (jnp.arange(pad, dtype=jnp.int32) % PADROWS)
    src = jnp.concatenate([ei[0], pad_idx]).reshape(NW, CPT, CHUNK)
    dst = jnp.concatenate([ei[1], pad_idx]).reshape(NW, CPT, CHUNK)
    return src, dst


def kernel(x1, x2, edge_index1, edge_index2, batch1, batch2, W1, b1, W2, b2,
           W3, b3, Watt, Wt, Wb, bt, Wfc, bfc, Wsc, bsc):
    f32 = jnp.float32
    xp1, xp2 = _pad_rows(x1), _pad_rows(x2)
    s1, d1 = _prep_edges(edge_index1)
    s2, d2 = _prep_edges(edge_index2)
    zvec = jnp.zeros((NPAD,), f32)
    ones_chunk = jnp.ones((CHUNK,), f32)

    dp1, dp2 = _deg_kernel()(d1, d2, zvec, ones_chunk)

    def prep1(xp, dp):
        return pl.pallas_call(
            _prep1_body,
            out_shape=(jax.ShapeDtypeStruct((NPAD, F1), f32),
                       jax.ShapeDtypeStruct((NPAD, 1), f32)),
        )(xp, dp.T, W1)

    hn1, dv1 = prep1(xp1, dp1)
    hn2, dv2 = prep1(xp2, dp2)

    a1p, a1p2 = _make_scatter(F1)(hn1, s1, d1, hn2, s2, d2,
                                  jnp.zeros((NPAD, F1), f32))

    def prep_mid(ap, hn, dv, b, W, Fo):
        return pl.pallas_call(
            _prep_mid_body,
            out_shape=jax.ShapeDtypeStruct((NPAD, Fo), f32),
        )(ap, hn, dv, b, W)

    hnb1 = prep_mid(a1p, hn1, dv1, b1.reshape(1, F1), W2, F2)
    hnb2 = prep_mid(a1p2, hn2, dv2, b1.reshape(1, F1), W2, F2)

    a2p, a2p2 = _make_scatter(F2)(hnb1, s1, d1, hnb2, s2, d2,
                                  jnp.zeros((NPAD, F2), f32))

    hnc1 = prep_mid(a2p, hnb1, dv1, b2.reshape(1, F2), W3, F3)
    hnc2 = prep_mid(a2p2, hnb2, dv2, b2.reshape(1, F2), W3, F3)

    a3p, a3p2 = _make_scatter(F3)(hnc1, s1, d1, hnc2, s2, d2,
                                  jnp.zeros((NPAD, F3), f32))

    mask = jnp.concatenate([jnp.ones((NN,), f32),
                            jnp.zeros((NPAD - NN,), f32)]).reshape(NPAD, 1)

    def pool(ap, hn, dv):
        return pl.pallas_call(
            _pool_body,
            out_shape=jax.ShapeDtypeStruct((1, KK), f32),
        )(ap, hn, dv, mask, b3.reshape(1, F3), Watt)

    p1 = pool(a3p, hnc1, dv1)
    p2 = pool(a3p2, hnc2, dv2)

    res = pl.pallas_call(
        _head_body,
        out_shape=jax.ShapeDtypeStruct((1, 1), f32),
    )(p1, p2, Wt.reshape(KK, KK * KK), Wb, bt.reshape(1, KK), Wfc,
      bfc.reshape(1, KK), Wsc, bsc.reshape(1, 1))
    return res.reshape(-1)


# speedup vs baseline: 47.0666x; 1.2301x over previous
import jax
import jax.numpy as jnp
from jax import lax
from jax.experimental import pallas as pl
from jax.experimental.pallas import tpu as pltpu
from jax.experimental.pallas import tpu_sc as plsc

NN = 10000
DD = 128
F1, F2, F3 = 64, 32, 16
KK = 16

NC, NS = 2, 16
NW = NC * NS
CHUNK = 128
NBUF = 8
GROUPS = 10
CPT = NBUF * GROUPS
EE = 320000
EPAD = NW * CPT * CHUNK
NPAD = 10240
RPT = NPAD // NS
PADROWS = 128


import functools


@functools.cache
def _mesh():
    return plsc.VectorSubcoreMesh(core_axis_name="c", subcore_axis_name="s",
                                  num_cores=NC, num_subcores=NS)


DNBUF = 4
DGROUPS = CPT // DNBUF


def _deg_body(dst1, dst2, zeros_h, ones_h, dp1, dp2,
              deg1, deg2, idxv, onesv, m0, m1, m2, m3):
    sems = [m0, m1, m2, m3]
    c = lax.axis_index("c")
    s = lax.axis_index("s")
    w = s * NC + c
    row0 = s * RPT
    pltpu.sync_copy(zeros_h.at[pl.ds(row0, RPT)], deg1.at[pl.ds(row0, RPT)])
    pltpu.sync_copy(zeros_h.at[pl.ds(row0, RPT)], deg2.at[pl.ds(row0, RPT)])
    pltpu.sync_copy(ones_h, onesv)
    plsc.subcore_barrier()
    for dsth, degv in ((dst1, deg1), (dst2, deg2)):
        pltpu.sync_copy(dsth.at[w], idxv)

        @pl.loop(0, DGROUPS)
        def _grp(g):
            for b in range(DNBUF):
                j = g * DNBUF + b

                @pl.when(g > 0)
                def _drain(b=b):
                    pltpu.make_async_copy(
                        onesv, degv.at[idxv.at[0]], sems[b]).wait()

                pltpu.async_copy(onesv, degv.at[idxv.at[j]], sems[b],
                                 add=True)

        for b in range(DNBUF):
            pltpu.make_async_copy(onesv, degv.at[idxv.at[0]], sems[b]).wait()
    plsc.subcore_barrier()
    pltpu.sync_copy(deg1.at[pl.ds(row0, RPT)], dp1.at[c, pl.ds(row0, RPT)])
    pltpu.sync_copy(deg2.at[pl.ds(row0, RPT)], dp2.at[c, pl.ds(row0, RPT)])


@functools.cache
def _deg_kernel():
    return pl.kernel(
        _deg_body,
        out_type=(jax.ShapeDtypeStruct((NC, NPAD), jnp.float32),) * 2,
        mesh=_mesh(),
        scratch_types=[
            pltpu.VMEM_SHARED((NPAD,), jnp.float32),
            pltpu.VMEM_SHARED((NPAD,), jnp.float32),
            pltpu.VMEM((CPT, CHUNK), jnp.int32),
            pltpu.VMEM((CHUNK,), jnp.float32),
        ] + [pltpu.SemaphoreType.DMA] * DNBUF,
    )


@functools.cache
def _make_scatter(F):
    def body(hn, srch, dsth, zeros_h, out,
             acc, srcv, dstv, *bs):
        bufs = list(bs[:NBUF])
        gsems = list(bs[NBUF:2 * NBUF])
        ssems = list(bs[2 * NBUF:])
        c = lax.axis_index("c")
        s = lax.axis_index("s")
        w = s * NC + c
        row0 = s * RPT
        pltpu.sync_copy(zeros_h.at[pl.ds(row0, RPT)],
                        acc.at[pl.ds(row0, RPT)])
        pltpu.sync_copy(srch.at[w], srcv)
        pltpu.sync_copy(dsth.at[w], dstv)
        plsc.subcore_barrier()
        for b in range(NBUF):
            pltpu.async_copy(hn.at[srcv.at[b]], bufs[b], gsems[b])

        @pl.loop(0, GROUPS)
        def _grp(g):
            for b in range(NBUF):
                j = g * NBUF + b
                pltpu.make_async_copy(
                    hn.at[srcv.at[j]], bufs[b], gsems[b]).wait()
                pltpu.async_copy(bufs[b], acc.at[dstv.at[j]], ssems[b],
                                 add=True)

                @pl.when(g < GROUPS - 1)
                def _refill(b=b, j=j):
                    pltpu.make_async_copy(
                        bufs[b], acc.at[dstv.at[j]], ssems[b]).wait()
                    pltpu.async_copy(hn.at[srcv.at[j + NBUF]], bufs[b],
                                     gsems[b])

        for b in range(NBUF):
            jl = CPT - NBUF + b
            pltpu.make_async_copy(
                bufs[b], acc.at[dstv.at[jl]], ssems[b]).wait()
        plsc.subcore_barrier()
        pltpu.sync_copy(acc.at[pl.ds(row0, RPT)],
                        out.at[c, pl.ds(row0, RPT)])

    return pl.kernel(
        body,
        out_type=jax.ShapeDtypeStruct((NC, NPAD, F), jnp.float32),
        mesh=_mesh(),
        scratch_types=[
            pltpu.VMEM_SHARED((NPAD, F), jnp.float32),
            pltpu.VMEM((CPT, CHUNK), jnp.int32),
            pltpu.VMEM((CPT, CHUNK), jnp.int32),
        ] + [pltpu.VMEM((CHUNK, F), jnp.float32)] * NBUF
          + [pltpu.SemaphoreType.DMA] * (2 * NBUF),
        compiler_params=pltpu.CompilerParams(use_tc_tiling_on_sc=False),
    )


def _prep1_body(x, dt, W1_in, hn_o, dv_o):
    deg = dt[:, 0:1] + dt[:, 1:2] + 1.0
    dinv = lax.rsqrt(deg)
    h = jnp.dot(x[...], W1_in[...], preferred_element_type=jnp.float32,
                precision=lax.Precision.HIGHEST)
    hn_o[...] = h * dinv
    dv_o[...] = dinv


def _prep_mid_body(ap, hn, dv, b_in, W_in, o):
    acc = jnp.sum(ap[...], axis=0)
    z = dv[...] * (acc + hn[...]) + b_in[...]
    a = jnp.maximum(z, 0.0)
    h = jnp.dot(a, W_in[...], preferred_element_type=jnp.float32,
                precision=lax.Precision.HIGHEST)
    o[...] = h * dv[...]


def _pool_body(ap, hn, dv, mask, b3_in, Watt_in, p_o):
    z = dv[...] * (jnp.sum(ap[...], axis=0) + hn[...]) + b3_in[...]
    zm = z * mask[...]
    mean = jnp.sum(zm, axis=0, keepdims=True) * (1.0 / NN)
    tg = jnp.tanh(jnp.dot(mean, Watt_in[...],
                          preferred_element_type=jnp.float32,
                          precision=lax.Precision.HIGHEST))
    logits = lax.dot_general(z, tg, (((1,), (1,)), ((), ())),
                             preferred_element_type=jnp.float32,
                             precision=lax.Precision.HIGHEST)
    coefs = jax.nn.sigmoid(logits)
    p_o[...] = jnp.sum(coefs * zm, axis=0, keepdims=True)


def _head_body(p1_in, p2_in, Wtr_in, Wb_in, bt_in, Wfc_in, bfc_in, Wsc_in,
               bsc_in, out):
    p1 = p1_in[...]
    p2 = p2_in[...]
    Wtr = Wtr_in[...]
    scoring = jnp.zeros((1, KK), jnp.float32)
    for j in range(KK):
        wtj = Wtr[:, j * KK:(j + 1) * KK]
        scoring = scoring + jnp.dot(
            p1, wtj, preferred_element_type=jnp.float32,
            precision=lax.Precision.HIGHEST) * p2[:, j:j + 1]
    comb = jnp.concatenate([p1, p2], axis=1)
    block = lax.dot_general(comb, Wb_in[...], (((1,), (1,)), ((), ())),
                            preferred_element_type=jnp.float32,
                            precision=lax.Precision.HIGHEST)
    sc = jnp.maximum(scoring + block + bt_in[...], 0.0)
    s2 = jnp.maximum(jnp.dot(sc, Wfc_in[...],
                             preferred_element_type=jnp.float32,
                             precision=lax.Precision.HIGHEST)
                     + bfc_in[...], 0.0)
    out[...] = jax.nn.sigmoid(jnp.dot(s2, Wsc_in[...],
                                      preferred_element_type=jnp.float32,
                                      precision=lax.Precision.HIGHEST)
                              + bsc_in[...])


def _pad_rows(x):
    return jnp.concatenate(
        [x, jnp.zeros((NPAD - NN, x.shape[1]), x.dtype)], axis=0)


def _prep_edges(ei):
    pad = EPAD - EE
    pad_idx = NN + (jnp.arange(pad, dtype=jnp.int32) % PADROWS)
    src = jnp.concatenate([ei[0], pad_idx]).reshape(NW, CPT, CHUNK)
    dst = jnp.concatenate([ei[1], pad_idx]).reshape(NW, CPT, CHUNK)
    return src, dst


def kernel(x1, x2, edge_index1, edge_index2, batch1, batch2, W1, b1, W2, b2,
           W3, b3, Watt, Wt, Wb, bt, Wfc, bfc, Wsc, bsc):
    f32 = jnp.float32
    xp1, xp2 = _pad_rows(x1), _pad_rows(x2)
    s1, d1 = _prep_edges(edge_index1)
    s2, d2 = _prep_edges(edge_index2)
    zvec = jnp.zeros((NPAD,), f32)
    ones_chunk = jnp.ones((CHUNK,), f32)

    dp1, dp2 = _deg_kernel()(d1, d2, zvec, ones_chunk)

    def prep1(xp, dp):
        return pl.pallas_call(
            _prep1_body,
            out_shape=(jax.ShapeDtypeStruct((NPAD, F1), f32),
                       jax.ShapeDtypeStruct((NPAD, 1), f32)),
        )(xp, dp.T, W1)

    hn1, dv1 = prep1(xp1, dp1)
    hn2, dv2 = prep1(xp2, dp2)

    z64 = jnp.zeros((NPAD, F1), f32)
    a1p = _make_scatter(F1)(hn1, s1, d1, z64)
    a1p2 = _make_scatter(F1)(hn2, s2, d2, z64)

    def prep_mid(ap, hn, dv, b, W, Fo):
        return pl.pallas_call(
            _prep_mid_body,
            out_shape=jax.ShapeDtypeStruct((NPAD, Fo), f32),
        )(ap, hn, dv, b, W)

    hnb1 = prep_mid(a1p, hn1, dv1, b1.reshape(1, F1), W2, F2)
    hnb2 = prep_mid(a1p2, hn2, dv2, b1.reshape(1, F1), W2, F2)

    z32 = jnp.zeros((NPAD, F2), f32)
    a2p = _make_scatter(F2)(hnb1, s1, d1, z32)
    a2p2 = _make_scatter(F2)(hnb2, s2, d2, z32)

    hnc1 = prep_mid(a2p, hnb1, dv1, b2.reshape(1, F2), W3, F3)
    hnc2 = prep_mid(a2p2, hnb2, dv2, b2.reshape(1, F2), W3, F3)

    z16 = jnp.zeros((NPAD, F3), f32)
    a3p = _make_scatter(F3)(hnc1, s1, d1, z16)
    a3p2 = _make_scatter(F3)(hnc2, s2, d2, z16)

    mask = jnp.concatenate([jnp.ones((NN,), f32),
                            jnp.zeros((NPAD - NN,), f32)]).reshape(NPAD, 1)

    def pool(ap, hn, dv):
        return pl.pallas_call(
            _pool_body,
            out_shape=jax.ShapeDtypeStruct((1, KK), f32),
        )(ap, hn, dv, mask, b3.reshape(1, F3), Watt)

    p1 = pool(a3p, hnc1, dv1)
    p2 = pool(a3p2, hnc2, dv2)

    res = pl.pallas_call(
        _head_body,
        out_shape=jax.ShapeDtypeStruct((1, 1), f32),
    )(p1, p2, Wt.reshape(KK, KK * KK), Wb, bt.reshape(1, KK), Wfc,
      bfc.reshape(1, KK), Wsc, bsc.reshape(1, 1))
    return res.reshape(-1)


# speedup vs baseline: 47.6575x; 1.0126x over previous
import jax
import jax.numpy as jnp
from jax import lax
from jax.experimental import pallas as pl
from jax.experimental.pallas import tpu as pltpu
from jax.experimental.pallas import tpu_sc as plsc

NN = 10000
DD = 128
F1, F2, F3 = 64, 32, 16
KK = 16

NC, NS = 2, 16
NW = NC * NS
CHUNK = 128
NBUF = 8
GROUPS = 10
CPT = NBUF * GROUPS
EE = 320000
EPAD = NW * CPT * CHUNK
NPAD = 10240
RPT = NPAD // NS
PADROWS = 128


import functools


@functools.cache
def _mesh():
    return plsc.VectorSubcoreMesh(core_axis_name="c", subcore_axis_name="s",
                                  num_cores=NC, num_subcores=NS)


DNBUF = 4
DGROUPS = CPT // DNBUF


def _deg_body(dsth, zeros_h, ones_h, dp,
              degv, idxv, onesv, m0, m1, m2, m3):
    sems = [m0, m1, m2, m3]
    c = lax.axis_index("c")
    s = lax.axis_index("s")
    w = s * NC + c
    row0 = s * RPT
    pltpu.sync_copy(zeros_h.at[pl.ds(row0, RPT)], degv.at[pl.ds(row0, RPT)])
    pltpu.sync_copy(ones_h, onesv)
    pltpu.sync_copy(dsth.at[w], idxv)
    plsc.subcore_barrier()

    @pl.loop(0, DGROUPS)
    def _grp(g):
        for b in range(DNBUF):
            j = g * DNBUF + b

            @pl.when(g > 0)
            def _drain(b=b):
                pltpu.make_async_copy(
                    onesv, degv.at[idxv.at[0]], sems[b]).wait()

            pltpu.async_copy(onesv, degv.at[idxv.at[j]], sems[b],
                             add=True)

    for b in range(DNBUF):
        pltpu.make_async_copy(onesv, degv.at[idxv.at[0]], sems[b]).wait()
    plsc.subcore_barrier()
    pltpu.sync_copy(degv.at[pl.ds(row0, RPT)], dp.at[c, pl.ds(row0, RPT)])


@functools.cache
def _deg_kernel():
    return pl.kernel(
        _deg_body,
        out_type=jax.ShapeDtypeStruct((NC, NPAD), jnp.float32),
        mesh=_mesh(),
        scratch_types=[
            pltpu.VMEM_SHARED((NPAD,), jnp.float32),
            pltpu.VMEM((CPT, CHUNK), jnp.int32),
            pltpu.VMEM((CHUNK,), jnp.float32),
        ] + [pltpu.SemaphoreType.DMA] * DNBUF,
        compiler_params=pltpu.CompilerParams(use_tc_tiling_on_sc=False),
    )


@functools.cache
def _make_scatter(F):
    def body(hn, srch, dsth, zeros_h, out,
             acc, srcv, dstv, *bs):
        bufs = list(bs[:NBUF])
        gsems = list(bs[NBUF:2 * NBUF])
        ssems = list(bs[2 * NBUF:])
        c = lax.axis_index("c")
        s = lax.axis_index("s")
        w = s * NC + c
        row0 = s * RPT
        pltpu.sync_copy(zeros_h.at[pl.ds(row0, RPT)],
                        acc.at[pl.ds(row0, RPT)])
        pltpu.sync_copy(srch.at[w], srcv)
        pltpu.sync_copy(dsth.at[w], dstv)
        plsc.subcore_barrier()
        for b in range(NBUF):
            pltpu.async_copy(hn.at[srcv.at[b]], bufs[b], gsems[b])

        @pl.loop(0, GROUPS)
        def _grp(g):
            for b in range(NBUF):
                j = g * NBUF + b
                pltpu.make_async_copy(
                    hn.at[srcv.at[j]], bufs[b], gsems[b]).wait()
                pltpu.async_copy(bufs[b], acc.at[dstv.at[j]], ssems[b],
                                 add=True)

                @pl.when(g < GROUPS - 1)
                def _refill(b=b, j=j):
                    pltpu.make_async_copy(
                        bufs[b], acc.at[dstv.at[j]], ssems[b]).wait()
                    pltpu.async_copy(hn.at[srcv.at[j + NBUF]], bufs[b],
                                     gsems[b])

        for b in range(NBUF):
            jl = CPT - NBUF + b
            pltpu.make_async_copy(
                bufs[b], acc.at[dstv.at[jl]], ssems[b]).wait()
        plsc.subcore_barrier()
        pltpu.sync_copy(acc.at[pl.ds(row0, RPT)],
                        out.at[c, pl.ds(row0, RPT)])

    return pl.kernel(
        body,
        out_type=jax.ShapeDtypeStruct((NC, NPAD, F), jnp.float32),
        mesh=_mesh(),
        scratch_types=[
            pltpu.VMEM_SHARED((NPAD, F), jnp.float32),
            pltpu.VMEM((CPT, CHUNK), jnp.int32),
            pltpu.VMEM((CPT, CHUNK), jnp.int32),
        ] + [pltpu.VMEM((CHUNK, F), jnp.float32)] * NBUF
          + [pltpu.SemaphoreType.DMA] * (2 * NBUF),
        compiler_params=pltpu.CompilerParams(use_tc_tiling_on_sc=False),
    )


EROWS = EE // CHUNK
PROWS = EPAD // CHUNK - EROWS


def _edge_prep_body(e1, e2, s1_o, d1_o, s2_o, d2_o):
    padrow = NN + lax.broadcasted_iota(jnp.int32, (PROWS, CHUNK), 1)
    for e, s_o, d_o in ((e1, s1_o, d1_o), (e2, s2_o, d2_o)):
        s_o[0:EROWS] = e[0]
        s_o[EROWS:] = padrow
        d_o[0:EROWS] = e[1]
        d_o[EROWS:] = padrow


def _xw_body(x, W_in, h_o):
    h_o[...] = jnp.dot(x[...], W_in[...], preferred_element_type=jnp.float32,
                       precision=lax.Precision.HIGHEST)


def _norm_body(h, dt, hn_o, dv_o):
    deg = dt[:, 0:1] + dt[:, 1:2] + 1.0
    dinv = 1.0 / jnp.sqrt(deg)
    hn_o[...] = h[...] * dinv
    dv_o[...] = dinv


def _prep_mid_body(ap, hn, dv, b_in, W_in, o):
    acc = jnp.sum(ap[...], axis=0)
    z = dv[...] * (acc + hn[...]) + b_in[...]
    a = jnp.maximum(z, 0.0)
    h = jnp.dot(a, W_in[...], preferred_element_type=jnp.float32,
                precision=lax.Precision.HIGHEST)
    o[...] = h * dv[...]


def _pool_body(ap, hn, dv, mask, b3_in, Watt_in, p_o):
    z = dv[...] * (jnp.sum(ap[...], axis=0) + hn[...]) + b3_in[...]
    zm = z * mask[...]
    mean = jnp.sum(zm, axis=0, keepdims=True) * (1.0 / NN)
    tg = jnp.tanh(jnp.dot(mean, Watt_in[...],
                          preferred_element_type=jnp.float32,
                          precision=lax.Precision.HIGHEST))
    logits = lax.dot_general(z, tg, (((1,), (1,)), ((), ())),
                             preferred_element_type=jnp.float32,
                             precision=lax.Precision.HIGHEST)
    coefs = jax.nn.sigmoid(logits)
    p_o[...] = jnp.sum(coefs * zm, axis=0, keepdims=True)


def _head_body(p1_in, p2_in, Wtr_in, Wb_in, bt_in, Wfc_in, bfc_in, Wsc_in,
               bsc_in, out):
    p1 = p1_in[...]
    p2 = p2_in[...]
    Wtr = Wtr_in[...]
    scoring = jnp.zeros((1, KK), jnp.float32)
    for j in range(KK):
        wtj = Wtr[:, j * KK:(j + 1) * KK]
        scoring = scoring + jnp.dot(
            p1, wtj, preferred_element_type=jnp.float32,
            precision=lax.Precision.HIGHEST) * p2[:, j:j + 1]
    comb = jnp.concatenate([p1, p2], axis=1)
    block = lax.dot_general(comb, Wb_in[...], (((1,), (1,)), ((), ())),
                            preferred_element_type=jnp.float32,
                            precision=lax.Precision.HIGHEST)
    sc = jnp.maximum(scoring + block + bt_in[...], 0.0)
    s2 = jnp.maximum(jnp.dot(sc, Wfc_in[...],
                             preferred_element_type=jnp.float32,
                             precision=lax.Precision.HIGHEST)
                     + bfc_in[...], 0.0)
    out[...] = jax.nn.sigmoid(jnp.dot(s2, Wsc_in[...],
                                      preferred_element_type=jnp.float32,
                                      precision=lax.Precision.HIGHEST)
                              + bsc_in[...])


def _pad_rows(x):
    return jnp.concatenate(
        [x, jnp.zeros((NPAD - NN, x.shape[1]), x.dtype)], axis=0)


def _prep_edges(ei):
    pad = EPAD - EE
    pad_idx = NN + (jnp.arange(pad, dtype=jnp.int32) % PADROWS)
    src = jnp.concatenate([ei[0], pad_idx]).reshape(NW, CPT, CHUNK)
    dst = jnp.concatenate([ei[1], pad_idx]).reshape(NW, CPT, CHUNK)
    return src, dst


def kernel(x1, x2, edge_index1, edge_index2, batch1, batch2, W1, b1, W2, b2,
           W3, b3, Watt, Wt, Wb, bt, Wfc, bfc, Wsc, bsc):
    f32 = jnp.float32
    xp1, xp2 = _pad_rows(x1), _pad_rows(x2)
    er1 = edge_index1.reshape(2, EROWS, CHUNK)
    er2 = edge_index2.reshape(2, EROWS, CHUNK)
    es = jax.ShapeDtypeStruct((EPAD // CHUNK, CHUNK), jnp.int32)
    s1r, d1r, s2r, d2r = pl.pallas_call(
        _edge_prep_body, out_shape=(es, es, es, es))(er1, er2)
    s1 = s1r.reshape(NW, CPT, CHUNK)
    d1 = d1r.reshape(NW, CPT, CHUNK)
    s2 = s2r.reshape(NW, CPT, CHUNK)
    d2 = d2r.reshape(NW, CPT, CHUNK)
    zvec = jnp.zeros((NPAD,), f32)
    ones_chunk = jnp.ones((CHUNK,), f32)

    dp1 = _deg_kernel()(d1, zvec, ones_chunk)
    dp2 = _deg_kernel()(d2, zvec, ones_chunk)

    def xw(xp):
        return pl.pallas_call(
            _xw_body, out_shape=jax.ShapeDtypeStruct((NPAD, F1), f32),
        )(xp, W1)

    h1 = xw(xp1)
    h2 = xw(xp2)

    def norm(h, dp):
        return pl.pallas_call(
            _norm_body,
            out_shape=(jax.ShapeDtypeStruct((NPAD, F1), f32),
                       jax.ShapeDtypeStruct((NPAD, 1), f32)),
        )(h, dp.T)

    hn1, dv1 = norm(h1, dp1)
    hn2, dv2 = norm(h2, dp2)

    z64 = jnp.zeros((NPAD, F1), f32)
    a1p = _make_scatter(F1)(hn1, s1, d1, z64)
    a1p2 = _make_scatter(F1)(hn2, s2, d2, z64)

    def prep_mid(ap, hn, dv, b, W, Fo):
        return pl.pallas_call(
            _prep_mid_body,
            out_shape=jax.ShapeDtypeStruct((NPAD, Fo), f32),
        )(ap, hn, dv, b, W)

    hnb1 = prep_mid(a1p, hn1, dv1, b1.reshape(1, F1), W2, F2)
    hnb2 = prep_mid(a1p2, hn2, dv2, b1.reshape(1, F1), W2, F2)

    z32 = jnp.zeros((NPAD, F2), f32)
    a2p = _make_scatter(F2)(hnb1, s1, d1, z32)
    a2p2 = _make_scatter(F2)(hnb2, s2, d2, z32)

    hnc1 = prep_mid(a2p, hnb1, dv1, b2.reshape(1, F2), W3, F3)
    hnc2 = prep_mid(a2p2, hnb2, dv2, b2.reshape(1, F2), W3, F3)

    z16 = jnp.zeros((NPAD, F3), f32)
    a3p = _make_scatter(F3)(hnc1, s1, d1, z16)
    a3p2 = _make_scatter(F3)(hnc2, s2, d2, z16)

    mask = jnp.concatenate([jnp.ones((NN,), f32),
                            jnp.zeros((NPAD - NN,), f32)]).reshape(NPAD, 1)

    def pool(ap, hn, dv):
        return pl.pallas_call(
            _pool_body,
            out_shape=jax.ShapeDtypeStruct((1, KK), f32),
        )(ap, hn, dv, mask, b3.reshape(1, F3), Watt)

    p1 = pool(a3p, hnc1, dv1)
    p2 = pool(a3p2, hnc2, dv2)

    res = pl.pallas_call(
        _head_body,
        out_shape=jax.ShapeDtypeStruct((1, 1), f32),
    )(p1, p2, Wt.reshape(KK, KK * KK), Wb, bt.reshape(1, KK), Wfc,
      bfc.reshape(1, KK), Wsc, bsc.reshape(1, 1))
    return res.reshape(-1)


# speedup vs baseline: 48.2562x; 1.0126x over previous
import jax
import jax.numpy as jnp
from jax import lax
from jax.experimental import pallas as pl
from jax.experimental.pallas import tpu as pltpu
from jax.experimental.pallas import tpu_sc as plsc

NN = 10000
DD = 128
F1, F2, F3 = 64, 32, 16
KK = 16

NC, NS = 2, 16
NW = NC * NS
CHUNK = 128
NBUF = 8
GROUPS = 10
CPT = NBUF * GROUPS
EE = 320000
EPAD = NW * CPT * CHUNK
NPAD = 10240
RPT = NPAD // NS
PADROWS = 128


import functools


@functools.cache
def _mesh():
    return plsc.VectorSubcoreMesh(core_axis_name="c", subcore_axis_name="s",
                                  num_cores=NC, num_subcores=NS)


DNBUF = 4
DGROUPS = CPT // DNBUF


def _deg_body(dst1, dst2, zeros_h, ones_h, dp1, dp2,
              deg1, deg2, idxv, onesv, m0, m1, m2, m3):
    sems = [m0, m1, m2, m3]
    c = lax.axis_index("c")
    s = lax.axis_index("s")
    w = s * NC + c
    row0 = s * RPT
    pltpu.sync_copy(zeros_h.at[pl.ds(row0, RPT)], deg1.at[pl.ds(row0, RPT)])
    pltpu.sync_copy(zeros_h.at[pl.ds(row0, RPT)], deg2.at[pl.ds(row0, RPT)])
    pltpu.sync_copy(ones_h, onesv)
    plsc.subcore_barrier()
    for dsth, degv in ((dst1, deg1), (dst2, deg2)):
        pltpu.sync_copy(dsth.at[w], idxv)

        @pl.loop(0, DGROUPS)
        def _grp(g, degv=degv):
            for b in range(DNBUF):
                j = g * DNBUF + b

                @pl.when(g > 0)
                def _drain(b=b, degv=degv):
                    pltpu.make_async_copy(
                        onesv, degv.at[idxv.at[0]], sems[b]).wait()

                pltpu.async_copy(onesv, degv.at[idxv.at[j]], sems[b],
                                 add=True)

        for b in range(DNBUF):
            pltpu.make_async_copy(onesv, degv.at[idxv.at[0]], sems[b]).wait()
    plsc.subcore_barrier()
    pltpu.sync_copy(deg1.at[pl.ds(row0, RPT)], dp1.at[c, pl.ds(row0, RPT)])
    pltpu.sync_copy(deg2.at[pl.ds(row0, RPT)], dp2.at[c, pl.ds(row0, RPT)])


@functools.cache
def _deg_kernel():
    return pl.kernel(
        _deg_body,
        out_type=(jax.ShapeDtypeStruct((NC, NPAD), jnp.float32),) * 2,
        mesh=_mesh(),
        scratch_types=[
            pltpu.VMEM_SHARED((NPAD,), jnp.float32),
            pltpu.VMEM_SHARED((NPAD,), jnp.float32),
            pltpu.VMEM((CPT, CHUNK), jnp.int32),
            pltpu.VMEM((CHUNK,), jnp.float32),
        ] + [pltpu.SemaphoreType.DMA] * DNBUF,
        compiler_params=pltpu.CompilerParams(use_tc_tiling_on_sc=False),
    )


@functools.cache
def _make_scatter(F):
    def body(hn, srch, dsth, zeros_h, out,
             acc, srcv, dstv, *bs):
        bufs = list(bs[:NBUF])
        gsems = list(bs[NBUF:2 * NBUF])
        ssems = list(bs[2 * NBUF:])
        c = lax.axis_index("c")
        s = lax.axis_index("s")
        w = s * NC + c
        row0 = s * RPT
        pltpu.sync_copy(zeros_h.at[pl.ds(row0, RPT)],
                        acc.at[pl.ds(row0, RPT)])
        pltpu.sync_copy(srch.at[w], srcv)
        pltpu.sync_copy(dsth.at[w], dstv)
        plsc.subcore_barrier()
        for b in range(NBUF):
            pltpu.async_copy(hn.at[srcv.at[b]], bufs[b], gsems[b])

        @pl.loop(0, GROUPS)
        def _grp(g):
            for b in range(NBUF):
                j = g * NBUF + b
                pltpu.make_async_copy(
                    hn.at[srcv.at[j]], bufs[b], gsems[b]).wait()
                pltpu.async_copy(bufs[b], acc.at[dstv.at[j]], ssems[b],
                                 add=True)

                @pl.when(g < GROUPS - 1)
                def _refill(b=b, j=j):
                    pltpu.make_async_copy(
                        bufs[b], acc.at[dstv.at[j]], ssems[b]).wait()
                    pltpu.async_copy(hn.at[srcv.at[j + NBUF]], bufs[b],
                                     gsems[b])

        for b in range(NBUF):
            jl = CPT - NBUF + b
            pltpu.make_async_copy(
                bufs[b], acc.at[dstv.at[jl]], ssems[b]).wait()
        plsc.subcore_barrier()
        pltpu.sync_copy(acc.at[pl.ds(row0, RPT)],
                        out.at[c, pl.ds(row0, RPT)])

    return pl.kernel(
        body,
        out_type=jax.ShapeDtypeStruct((NC, NPAD, F), jnp.float32),
        mesh=_mesh(),
        scratch_types=[
            pltpu.VMEM_SHARED((NPAD, F), jnp.float32),
            pltpu.VMEM((CPT, CHUNK), jnp.int32),
            pltpu.VMEM((CPT, CHUNK), jnp.int32),
        ] + [pltpu.VMEM((CHUNK, F), jnp.float32)] * NBUF
          + [pltpu.SemaphoreType.DMA] * (2 * NBUF),
        compiler_params=pltpu.CompilerParams(use_tc_tiling_on_sc=False),
    )


EROWS = EE // CHUNK
PROWS = EPAD // CHUNK - EROWS


def _edge_prep_body(e1, e2, s1_o, d1_o, s2_o, d2_o):
    padrow = NN + lax.broadcasted_iota(jnp.int32, (PROWS, CHUNK), 1)
    for e, s_o, d_o in ((e1, s1_o, d1_o), (e2, s2_o, d2_o)):
        s_o[0:EROWS] = e[0]
        s_o[EROWS:] = padrow
        d_o[0:EROWS] = e[1]
        d_o[EROWS:] = padrow


def _xw_body(x, W_in, h_o):
    h_o[...] = jnp.dot(x[...], W_in[...], preferred_element_type=jnp.float32,
                       precision=lax.Precision.HIGHEST)


def _norm_body(h, dt, hn_o, dv_o):
    deg = dt[:, 0:1] + dt[:, 1:2] + 1.0
    dinv = 1.0 / jnp.sqrt(deg)
    hn_o[...] = h[...] * dinv
    dv_o[...] = dinv


def _prep_mid_body(ap, hn, dv, b_in, W_in, o):
    acc = jnp.sum(ap[...], axis=0)
    z = dv[...] * (acc + hn[...]) + b_in[...]
    a = jnp.maximum(z, 0.0)
    h = jnp.dot(a, W_in[...], preferred_element_type=jnp.float32,
                precision=lax.Precision.HIGHEST)
    o[...] = h * dv[...]


PP = NPAD * F3 // 128


def _pool_core(ap, hn, dv, mask, b3_in, Watt_in):
    acc = ap[0] + ap[1]
    z = dv[...] * (acc + hn[...]) + b3_in[...]
    zm = z * mask[...]
    mean = jnp.sum(zm, axis=0, keepdims=True) * (1.0 / NN)
    tg = jnp.tanh(jnp.dot(mean, Watt_in[...],
                          preferred_element_type=jnp.float32,
                          precision=lax.Precision.HIGHEST))
    logits = lax.dot_general(z, tg, (((1,), (1,)), ((), ())),
                             preferred_element_type=jnp.float32,
                             precision=lax.Precision.HIGHEST)
    coefs = jax.nn.sigmoid(logits)
    return jnp.sum(coefs * zm, axis=0, keepdims=True)


def _pool_body(ap, hn, dv, mask, b3_in, Watt_in, p_o):
    p_o[...] = _pool_core(ap[...], hn, dv, mask, b3_in, Watt_in)


def _pool_head_body(ap, hn, dv, mask, b3_in, Watt_in, p1_in, Wtr_in, Wb_in,
                    bt_in, Wfc_in, bfc_in, Wsc_in, bsc_in, out):
    p2 = _pool_core(ap[...], hn, dv, mask, b3_in, Watt_in)
    p1 = p1_in[...]
    Wtr = Wtr_in[...]
    scoring = jnp.zeros((1, KK), jnp.float32)
    for j in range(KK):
        wtj = Wtr[:, j * KK:(j + 1) * KK]
        scoring = scoring + jnp.dot(
            p1, wtj, preferred_element_type=jnp.float32,
            precision=lax.Precision.HIGHEST) * p2[:, j:j + 1]
    comb = jnp.concatenate([p1, p2], axis=1)
    block = lax.dot_general(comb, Wb_in[...], (((1,), (1,)), ((), ())),
                            preferred_element_type=jnp.float32,
                            precision=lax.Precision.HIGHEST)
    sc = jnp.maximum(scoring + block + bt_in[...], 0.0)
    s2 = jnp.maximum(jnp.dot(sc, Wfc_in[...],
                             preferred_element_type=jnp.float32,
                             precision=lax.Precision.HIGHEST)
                     + bfc_in[...], 0.0)
    out[...] = jax.nn.sigmoid(jnp.dot(s2, Wsc_in[...],
                                      preferred_element_type=jnp.float32,
                                      precision=lax.Precision.HIGHEST)
                              + bsc_in[...])


def _pad_rows(x):
    return jnp.concatenate(
        [x, jnp.zeros((NPAD - NN, x.shape[1]), x.dtype)], axis=0)


def _prep_edges(ei):
    pad = EPAD - EE
    pad_idx = NN + (jnp.arange(pad, dtype=jnp.int32) % PADROWS)
    src = jnp.concatenate([ei[0], pad_idx]).reshape(NW, CPT, CHUNK)
    dst = jnp.concatenate([ei[1], pad_idx]).reshape(NW, CPT, CHUNK)
    return src, dst


def kernel(x1, x2, edge_index1, edge_index2, batch1, batch2, W1, b1, W2, b2,
           W3, b3, Watt, Wt, Wb, bt, Wfc, bfc, Wsc, bsc):
    f32 = jnp.float32
    xp1, xp2 = _pad_rows(x1), _pad_rows(x2)
    er1 = edge_index1.reshape(2, EROWS, CHUNK)
    er2 = edge_index2.reshape(2, EROWS, CHUNK)
    es = jax.ShapeDtypeStruct((EPAD // CHUNK, CHUNK), jnp.int32)
    s1r, d1r, s2r, d2r = pl.pallas_call(
        _edge_prep_body, out_shape=(es, es, es, es))(er1, er2)
    s1 = s1r.reshape(NW, CPT, CHUNK)
    d1 = d1r.reshape(NW, CPT, CHUNK)
    s2 = s2r.reshape(NW, CPT, CHUNK)
    d2 = d2r.reshape(NW, CPT, CHUNK)
    zvec = jnp.zeros((NPAD,), f32)
    ones_chunk = jnp.ones((CHUNK,), f32)

    dp1, dp2 = _deg_kernel()(d1, d2, zvec, ones_chunk)

    def xw(xp):
        return pl.pallas_call(
            _xw_body, out_shape=jax.ShapeDtypeStruct((NPAD, F1), f32),
        )(xp, W1)

    h1 = xw(xp1)
    h2 = xw(xp2)

    def norm(h, dp):
        return pl.pallas_call(
            _norm_body,
            out_shape=(jax.ShapeDtypeStruct((NPAD, F1), f32),
                       jax.ShapeDtypeStruct((NPAD, 1), f32)),
        )(h, dp.T)

    hn1, dv1 = norm(h1, dp1)
    hn2, dv2 = norm(h2, dp2)

    z64 = jnp.zeros((NPAD, F1), f32)
    a1p = _make_scatter(F1)(hn1, s1, d1, z64)
    a1p2 = _make_scatter(F1)(hn2, s2, d2, z64)

    def prep_mid(ap, hn, dv, b, W, Fo):
        return pl.pallas_call(
            _prep_mid_body,
            out_shape=jax.ShapeDtypeStruct((NPAD, Fo), f32),
        )(ap, hn, dv, b, W)

    hnb1 = prep_mid(a1p, hn1, dv1, b1.reshape(1, F1), W2, F2)
    hnb2 = prep_mid(a1p2, hn2, dv2, b1.reshape(1, F1), W2, F2)

    z32 = jnp.zeros((NPAD, F2), f32)
    a2p = _make_scatter(F2)(hnb1, s1, d1, z32)
    a2p2 = _make_scatter(F2)(hnb2, s2, d2, z32)

    hnc1 = prep_mid(a2p, hnb1, dv1, b2.reshape(1, F2), W3, F3)
    hnc2 = prep_mid(a2p2, hnb2, dv2, b2.reshape(1, F2), W3, F3)

    z16 = jnp.zeros((NPAD, F3), f32)
    a3p = _make_scatter(F3)(hnc1, s1, d1, z16)
    a3p2 = _make_scatter(F3)(hnc2, s2, d2, z16)

    mask = jnp.concatenate([jnp.ones((NN,), f32),
                            jnp.zeros((NPAD - NN,), f32)]).reshape(NPAD, 1)

    p1 = pl.pallas_call(
        _pool_body, out_shape=jax.ShapeDtypeStruct((1, KK), f32),
    )(a3p, hnc1, dv1, mask, b3.reshape(1, F3), Watt)

    res = pl.pallas_call(
        _pool_head_body, out_shape=jax.ShapeDtypeStruct((1, 1), f32),
    )(a3p2, hnc2, dv2, mask, b3.reshape(1, F3), Watt, p1,
      Wt.reshape(KK, KK * KK), Wb, bt.reshape(1, KK), Wfc,
      bfc.reshape(1, KK), Wsc, bsc.reshape(1, 1))
    return res.reshape(-1)
